# Initial kernel scaffold; baseline (speedup 1.0000x reference)
import jax
import jax.numpy as jnp
from jax.experimental import pallas as pl


def kernel(features):
    raise NotImplementedError("write your pallas kernel here")


# speedup vs baseline: 11.1073x; 11.1073x over previous
import functools

import jax
import jax.numpy as jnp
from jax import lax
from jax.experimental import pallas as pl
from jax.experimental.pallas import tpu as pltpu
from jax.experimental.pallas import tpu_sc as plsc

KTOP = 1024
NC, NS, L = 2, 16, 16
NW = NC * NS
NROW, ROWLEN = 64, 32 * 4096
ROWS_PER_W = NROW // NW
CH = 16384
NCHUNK = ROWLEN // CH
NV = CH // L

_mesh = plsc.VectorSubcoreMesh(
    core_axis_name="c", subcore_axis_name="s", num_cores=NC, num_subcores=NS
)


def _ukey(v):
  u = lax.bitcast_convert_type(v, jnp.uint32)
  s = u >> jnp.uint32(31)
  m = (s ^ jnp.uint32(1)) * jnp.uint32(0x7FFFFFFF)
  return u ^ m


@functools.partial(
    pl.kernel,
    out_type=jax.ShapeDtypeStruct((NROW, ROWLEN), jnp.float32),
    mesh=_mesh,
    scratch_types=[
        pltpu.VMEM((CH,), jnp.float32),
        pltpu.VMEM((CH,), jnp.float32),
        pltpu.VMEM((L * 4096,), jnp.int32),
    ],
    compiler_params=pltpu.CompilerParams(needs_layout_passes=False),
)
def _topk_mask(x_hbm, out_hbm, buf0, buf1, hist):
  wid = lax.axis_index("c") * NS + lax.axis_index("s")
  lane = lax.iota(jnp.int32, 16)
  ones = jnp.ones((L,), jnp.int32)

  def zero_hist(nwords):
    zv = jnp.zeros((L,), jnp.int32)

    @pl.loop(0, nwords // L, unroll=8)
    def _(i):
      hist[pl.ds(i * L, L)] = zv

  def hist_pass(row, nbins, bin_fn, mask_fn):
    lane_base = lane * nbins

    @pl.loop(0, NCHUNK)
    def _(ci):
      pltpu.sync_copy(x_hbm.at[row, pl.ds(ci * CH, CH)], buf0)

      @pl.loop(0, NV, unroll=4)
      def _(i):
        uk = _ukey(buf0[pl.ds(i * L, L)])
        addr = lane_base + lax.bitcast_convert_type(bin_fn(uk), jnp.int32)
        plsc.addupdate_scatter(hist, [addr], ones, mask=mask_fn(uk))

  def scan_level(nbins, kt):
    def body(g, carry):
      cumtot, bcnt, cumbefore = carry
      tot = hist[pl.ds(g * L, L)]
      for l in range(1, L):
        tot = tot + hist[pl.ds(l * nbins + g * L, L)]
      cum = plsc.cumsum(tot) + cumtot
      m = cum < kt
      bcnt = bcnt + jnp.sum(m.astype(jnp.int32))
      cumbefore = jnp.maximum(cumbefore, jnp.max(jnp.where(m, cum, 0)))
      cumtot = cumtot + jnp.sum(tot)
      return cumtot, bcnt, cumbefore

    z = jnp.int32(0)
    _, b, cb = pl.loop(0, nbins // L, init_carry=(z, z, z))(body)
    return b, kt - cb

  def process_row(row):
    zero_hist(L * 4096)
    hist_pass(row, 4096,
              lambda uk: uk >> jnp.uint32(20),
              lambda uk: None)
    b1, k1 = scan_level(4096, jnp.int32(KTOP))

    zero_hist(L * 4096)
    hist_pass(row, 4096,
              lambda uk: (uk >> jnp.uint32(8)) & jnp.uint32(0xFFF),
              lambda uk: lax.bitcast_convert_type(uk >> jnp.uint32(20), jnp.int32) == b1)
    b2, k2 = scan_level(4096, k1)

    zero_hist(L * 256)
    hist_pass(row, 256,
              lambda uk: uk & jnp.uint32(0xFF),
              lambda uk: (lax.bitcast_convert_type(uk >> jnp.uint32(20), jnp.int32) == b1)
              & (lax.bitcast_convert_type((uk >> jnp.uint32(8)) & jnp.uint32(0xFFF),
                              jnp.int32) == b2))
    b3, _ = scan_level(256, k2)

    t = (b1 << 20) | (b2 << 8) | b3
    mono = ~t
    sgn = lax.shift_right_logical(mono, 31)
    ut = jnp.where(sgn == 1, mono ^ jnp.int32(-(2 ** 31)), ~mono)
    thrv = lax.bitcast_convert_type(jnp.broadcast_to(ut, (L,)), jnp.float32)

    @pl.loop(0, NCHUNK)
    def _(ci):
      pltpu.sync_copy(x_hbm.at[row, pl.ds(ci * CH, CH)], buf0)

      @pl.loop(0, NV, unroll=4)
      def _(i):
        v = buf0[pl.ds(i * L, L)]
        buf1[pl.ds(i * L, L)] = jnp.where(v >= thrv, v, jnp.float32(0.0))

      pltpu.sync_copy(buf1, out_hbm.at[row, pl.ds(ci * CH, CH)])

  for r in range(ROWS_PER_W):
    process_row(wid * ROWS_PER_W + r)


def kernel(features):
  b, nl, d = features.shape
  out = _topk_mask(features.reshape(b, nl * d))
  return out.reshape(b, nl, d)


# speedup vs baseline: 37.6176x; 3.3868x over previous
import functools

import jax
import jax.numpy as jnp
from jax import lax
from jax.experimental import pallas as pl
from jax.experimental.pallas import tpu as pltpu
from jax.experimental.pallas import tpu_sc as plsc

KTOP = 1024
NC, NS, L = 2, 16, 16
NW = NC * NS
NROW, ROWLEN = 64, 32 * 4096
ROWS_PER_W = NROW // NW
NBUF = 4
CH = 8192
NCHUNK = ROWLEN // CH
NV = CH // L

_mesh = plsc.VectorSubcoreMesh(
    core_axis_name="c", subcore_axis_name="s", num_cores=NC, num_subcores=NS
)


def _ukey(v):
  u = lax.bitcast_convert_type(v, jnp.uint32)
  s = u >> jnp.uint32(31)
  m = (s ^ jnp.uint32(1)) * jnp.uint32(0x7FFFFFFF)
  return u ^ m


@functools.partial(
    pl.kernel,
    out_type=jax.ShapeDtypeStruct((NROW, ROWLEN), jnp.float32),
    mesh=_mesh,
    scratch_types=[
        [pltpu.VMEM((CH,), jnp.float32) for _ in range(NBUF)],
        pltpu.VMEM((L * 4096,), jnp.int32),
        pltpu.VMEM((4096,), jnp.int32),
        pltpu.SMEM((256,), jnp.int32),
        [pltpu.SemaphoreType.DMA for _ in range(NBUF)],
        [pltpu.SemaphoreType.DMA for _ in range(NBUF)],
    ],
    compiler_params=pltpu.CompilerParams(needs_layout_passes=False),
)
def _topk_mask(x_hbm, out_hbm, bufs, hist, totbuf, gsum, sin, sout):
  lane = lax.iota(jnp.int32, 16)
  ones = jnp.ones((L,), jnp.int32)

  def in_copy(row, c, k):
    return pltpu.make_async_copy(
        x_hbm.at[row, pl.ds(c * CH, CH)], bufs[k], sin[k])

  def out_copy(row, c, k):
    return pltpu.make_async_copy(
        bufs[k], out_hbm.at[row, pl.ds(c * CH, CH)], sout[k])

  def stream_pass(row, compute_chunk, writeback):
    for k in range(NBUF - 1):
      in_copy(row, k, k).start()

    @pl.loop(0, NCHUNK // NBUF)
    def _(j):
      for k in range(NBUF):
        c = j * NBUF + k
        in_copy(row, c, k).wait()
        compute_chunk(bufs[k])
        if writeback:
          out_copy(row, c, k).start()
        m = (k + NBUF - 1) % NBUF
        if writeback:
          @pl.when((c + NBUF - 1 < NCHUNK) & (c > 0))
          def _():
            out_copy(row, c - 1, m).wait()

        @pl.when(c + NBUF - 1 < NCHUNK)
        def _():
          in_copy(row, c + NBUF - 1, m).start()

    if writeback:
      for k in range(NBUF):
        out_copy(row, NCHUNK - NBUF + k, k).wait()

  def zero_hist(nwords):
    zv = jnp.zeros((L,), jnp.int32)

    @plsc.parallel_loop(0, nwords // L, unroll=8)
    def _(i):
      hist[pl.ds(i * L, L)] = zv

  def hist_pass(row, nbins, bin_fn, mask_fn):
    lane_base = lane * nbins

    def compute_chunk(buf):
      @plsc.parallel_loop(0, NV, unroll=8)
      def _(i):
        uk = _ukey(buf[pl.ds(i * L, L)])
        addr = lane_base + lax.bitcast_convert_type(bin_fn(uk), jnp.int32)
        plsc.addupdate_scatter(hist, [addr], ones, mask=mask_fn(uk))

    stream_pass(row, compute_chunk, writeback=False)

  def scan_level(nbins, kt):
    ngrp = nbins // L

    @plsc.parallel_loop(0, ngrp, unroll=2)
    def _(g):
      tot = hist[pl.ds(g * L, L)]
      for l in range(1, L):
        tot = tot + hist[pl.ds(l * nbins + g * L, L)]
      totbuf[pl.ds(g * L, L)] = tot
      gsum[g] = jnp.sum(tot)

    def gbody(g, carry):
      ct, gcnt, cb = carry
      nct = ct + gsum[g]
      below = (nct < kt).astype(jnp.int32)
      return nct, gcnt + below, jnp.where(below == 1, nct, cb)

    z = jnp.int32(0)
    _, gstar, cbg = pl.loop(0, ngrp, init_carry=(z, z, z))(gbody)

    cum = plsc.cumsum(totbuf[pl.ds(gstar * L, L)]) + cbg
    m = cum < kt
    b = gstar * L + jnp.sum(m.astype(jnp.int32))
    cb = jnp.maximum(cbg, jnp.max(jnp.where(m, cum, 0)))
    return b, kt - cb

  def process_row(row):
    zero_hist(L * 4096)
    hist_pass(row, 4096,
              lambda uk: uk >> jnp.uint32(20),
              lambda uk: None)
    b1, k1 = scan_level(4096, jnp.int32(KTOP))

    zero_hist(L * 4096)
    hist_pass(row, 4096,
              lambda uk: (uk >> jnp.uint32(8)) & jnp.uint32(0xFFF),
              lambda uk: lax.bitcast_convert_type(
                  uk >> jnp.uint32(20), jnp.int32) == b1)
    b2, k2 = scan_level(4096, k1)

    zero_hist(L * 256)
    hist_pass(row, 256,
              lambda uk: uk & jnp.uint32(0xFF),
              lambda uk: (lax.bitcast_convert_type(
                  uk >> jnp.uint32(20), jnp.int32) == b1)
              & (lax.bitcast_convert_type(
                  (uk >> jnp.uint32(8)) & jnp.uint32(0xFFF), jnp.int32) == b2))
    b3, _ = scan_level(256, k2)

    t = (b1 << 20) | (b2 << 8) | b3
    mono = ~t
    sgn = lax.shift_right_logical(mono, 31)
    ut = jnp.where(sgn == 1, mono ^ jnp.int32(-(2 ** 31)), ~mono)
    thrv = lax.bitcast_convert_type(
        jnp.broadcast_to(ut, (L,)), jnp.float32)

    def mask_chunk(buf):
      @plsc.parallel_loop(0, NV, unroll=8)
      def _(i):
        v = buf[pl.ds(i * L, L)]
        buf[pl.ds(i * L, L)] = jnp.where(v >= thrv, v, jnp.float32(0.0))

    stream_pass(row, mask_chunk, writeback=True)

  wid = lax.axis_index("c") * NS + lax.axis_index("s")
  for r in range(ROWS_PER_W):
    process_row(wid * ROWS_PER_W + r)


def kernel(features):
  b, nl, d = features.shape
  out = _topk_mask(features.reshape(b, nl * d))
  return out.reshape(b, nl, d)


# speedup vs baseline: 51.2474x; 1.3623x over previous
import functools

import jax
import jax.numpy as jnp
from jax import lax
from jax.experimental import pallas as pl
from jax.experimental.pallas import tpu as pltpu
from jax.experimental.pallas import tpu_sc as plsc

KTOP = 1024
NC, NS, L = 2, 16, 16
NW = NC * NS
NROW, NL, ND = 64, 32, 4096
ROWS_PER_W = NROW // NW
CL = 8
NCHUNK = NL // CL
NVL = ND // L

B1S, B2S, B3S = 21, 10, 0
NB1, NB2, NB3 = 2048, 2048, 1024

_mesh = plsc.VectorSubcoreMesh(
    core_axis_name="c", subcore_axis_name="s", num_cores=NC, num_subcores=NS
)


def _ukey(v):
  u = lax.bitcast_convert_type(v, jnp.int32)
  return jnp.where(u < 0, u, u ^ jnp.int32(0x7FFFFFFF))


@functools.partial(
    pl.kernel,
    out_type=jax.ShapeDtypeStruct((NROW, NL, ND), jnp.float32),
    mesh=_mesh,
    scratch_types=[
        [pltpu.VMEM((CL, ND), jnp.float32) for _ in range(2)],
        pltpu.VMEM((L * NB1,), jnp.int32),
        pltpu.VMEM((NB1,), jnp.int32),
        pltpu.SMEM((256,), jnp.int32),
        [pltpu.SemaphoreType.DMA for _ in range(2)],
        [pltpu.SemaphoreType.DMA for _ in range(2)],
    ],
    compiler_params=pltpu.CompilerParams(needs_layout_passes=False),
)
def _topk_mask(x_hbm, out_hbm, bufs, hist, totbuf, gsum, sin, sout):
  lane = lax.iota(jnp.int32, 16)
  ones = jnp.ones((L,), jnp.int32)

  def in_copy(row, c, k):
    return pltpu.make_async_copy(
        x_hbm.at[row, pl.ds(c * CL, CL)], bufs[k], sin[k])

  def out_copy(row, c, k):
    return pltpu.make_async_copy(
        bufs[k], out_hbm.at[row, pl.ds(c * CL, CL)], sout[k])

  def stream_pass(row, compute_chunk, writeback):
    in_copy(row, 0, 0).start()

    @pl.loop(0, NCHUNK // 2)
    def _(j):
      for k in range(2):
        c = 2 * j + k
        other = 1 - k
        if writeback:
          @pl.when(c > 0)
          def _():
            out_copy(row, c - 1, other).wait()

        @pl.when(c + 1 < NCHUNK)
        def _():
          in_copy(row, c + 1, other).start()

        in_copy(row, c, k).wait()
        compute_chunk(bufs[k])
        if writeback:
          out_copy(row, c, k).start()

    if writeback:
      out_copy(row, NCHUNK - 1, 1).wait()

  def zero_hist(nwords):
    zv = jnp.zeros((L,), jnp.int32)

    @plsc.parallel_loop(0, nwords // L, unroll=8)
    def _(i):
      hist[pl.ds(i * L, L)] = zv

  def hist_pass(row, nbins, bin_fn, mask_fn):
    lane_base = lane * nbins

    def compute_chunk(buf):
      for sub in range(CL):
        @plsc.parallel_loop(0, NVL, unroll=8)
        def _(i):
          uk = _ukey(buf[sub, pl.ds(i * L, L)])
          addr = lane_base + bin_fn(uk)
          plsc.addupdate_scatter(hist, [addr], ones, mask=mask_fn(uk))

    stream_pass(row, compute_chunk, writeback=False)

  def scan_level(nbins, kt):
    ngrp = nbins // L

    @plsc.parallel_loop(0, ngrp, unroll=2)
    def _(g):
      tot = hist[pl.ds(g * L, L)]
      for l in range(1, L):
        tot = tot + hist[pl.ds(l * nbins + g * L, L)]
      totbuf[pl.ds(g * L, L)] = tot
      gsum[g] = jnp.sum(tot)

    def gbody(g, carry):
      ct, gcnt, cb = carry
      nct = ct + gsum[g]
      below = (nct < kt).astype(jnp.int32)
      return nct, gcnt + below, jnp.where(below == 1, nct, cb)

    z = jnp.int32(0)
    _, gstar, cbg = pl.loop(0, ngrp, init_carry=(z, z, z))(gbody)

    cum = plsc.cumsum(totbuf[pl.ds(gstar * L, L)]) + cbg
    m = cum < kt
    b = gstar * L + jnp.sum(m.astype(jnp.int32))
    cb = jnp.maximum(cbg, jnp.max(jnp.where(m, cum, 0)))
    return b, kt - cb

  def process_row(row):
    shrl = lax.shift_right_logical

    zero_hist(L * NB1)
    hist_pass(row, NB1,
              lambda uk: shrl(uk, B1S),
              lambda uk: None)
    b1, k1 = scan_level(NB1, jnp.int32(KTOP))

    zero_hist(L * NB2)
    hist_pass(row, NB2,
              lambda uk: shrl(uk, B2S) & jnp.int32(NB2 - 1),
              lambda uk: shrl(uk, B1S) == b1)
    b2, k2 = scan_level(NB2, k1)

    p2 = (b1 << (B1S - B2S)) | b2
    zero_hist(L * NB3)
    hist_pass(row, NB3,
              lambda uk: uk & jnp.int32(NB3 - 1),
              lambda uk: shrl(uk, B2S) == p2)
    b3, _ = scan_level(NB3, k2)

    t = (b1 << B1S) | (b2 << B2S) | b3
    mono = ~t
    sgn = shrl(mono, 31)
    ut = jnp.where(sgn == 1, mono ^ jnp.int32(-(2 ** 31)), ~mono)
    thrv = lax.bitcast_convert_type(
        jnp.broadcast_to(ut, (L,)), jnp.float32)

    def mask_chunk(buf):
      for sub in range(CL):
        @plsc.parallel_loop(0, NVL, unroll=8)
        def _(i):
          v = buf[sub, pl.ds(i * L, L)]
          buf[sub, pl.ds(i * L, L)] = jnp.where(
              v >= thrv, v, jnp.float32(0.0))

    stream_pass(row, mask_chunk, writeback=True)

  wid = lax.axis_index("c") * NS + lax.axis_index("s")
  for r in range(ROWS_PER_W):
    process_row(wid * ROWS_PER_W + r)


def kernel(features):
  return _topk_mask(features)


# speedup vs baseline: 56.4615x; 1.1017x over previous
import functools

import jax
import jax.numpy as jnp
from jax import lax
from jax.experimental import pallas as pl
from jax.experimental.pallas import tpu as pltpu
from jax.experimental.pallas import tpu_sc as plsc

KTOP = 1024
NC, NS, L = 2, 16, 16
NW = NC * NS
NROW, NL, ND = 64, 32, 4096
ROWS_PER_W = NROW // NW
CL = 8
NCHUNK = NL // CL
NVL = ND // L

B1S, B2S = 21, 10
NB1, NB2, NB3 = 2048, 2048, 1024
CAND = 8192

_mesh = plsc.VectorSubcoreMesh(
    core_axis_name="c", subcore_axis_name="s", num_cores=NC, num_subcores=NS
)


def _ukey(v):
  u = lax.bitcast_convert_type(v, jnp.int32)
  return jnp.where(u < 0, u, u ^ jnp.int32(0x7FFFFFFF))


@functools.partial(
    pl.kernel,
    out_type=jax.ShapeDtypeStruct((NROW, NL, ND), jnp.float32),
    mesh=_mesh,
    scratch_types=[
        [pltpu.VMEM((CL, ND), jnp.float32) for _ in range(2)],
        pltpu.VMEM((L * (NB1 + 1),), jnp.int32),
        pltpu.VMEM((NB1,), jnp.int32),
        pltpu.VMEM((CAND,), jnp.int32),
        pltpu.SMEM((256,), jnp.int32),
        [pltpu.SemaphoreType.DMA for _ in range(2)],
        [pltpu.SemaphoreType.DMA for _ in range(2)],
    ],
    compiler_params=pltpu.CompilerParams(needs_layout_passes=False),
)
def _topk_mask(x_hbm, out_hbm, bufs, hist, totbuf, cand, gsum, sin, sout):
  lane = lax.iota(jnp.int32, 16)
  ones = jnp.ones((L,), jnp.int32)

  def in_copy(row, c, k):
    return pltpu.make_async_copy(
        x_hbm.at[row, pl.ds(c * CL, CL)], bufs[k], sin[k])

  def out_copy(row, c, k):
    return pltpu.make_async_copy(
        bufs[k], out_hbm.at[row, pl.ds(c * CL, CL)], sout[k])

  def stream_pass(row, compute_chunk, writeback=False, carry_init=None):
    in_copy(row, 0, 0).start()

    def jbody(j, carry):
      for k in range(2):
        c = 2 * j + k
        other = 1 - k
        if writeback:
          @pl.when(c > 0)
          def _():
            out_copy(row, c - 1, other).wait()

        @pl.when(c + 1 < NCHUNK)
        def _():
          in_copy(row, c + 1, other).start()

        in_copy(row, c, k).wait()
        carry = compute_chunk(bufs[k], carry)
        if writeback:
          out_copy(row, c, k).start()
      return carry

    if carry_init is None:
      pl.loop(0, NCHUNK // 2)(lambda j: jbody(j, None) and None)
      out = None
    else:
      out = pl.loop(0, NCHUNK // 2, init_carry=carry_init)(jbody)

    if writeback:
      out_copy(row, NCHUNK - 1, 1).wait()
    return out

  def zero_hist(nwords):
    zv = jnp.zeros((L,), jnp.int32)

    @plsc.parallel_loop(0, nwords // L, unroll=8)
    def _(i):
      hist[pl.ds(i * L, L)] = zv

  def scan_level(nbins, kt):
    ngrp = nbins // L

    @plsc.parallel_loop(0, ngrp, unroll=2)
    def _(g):
      tot = hist[pl.ds(g * L, L)]
      for l in range(1, L):
        tot = tot + hist[pl.ds(l * (nbins + 1) + g * L, L)]
      totbuf[pl.ds(g * L, L)] = tot
      gsum[g] = jnp.sum(tot)

    def gbody(g, carry):
      ct, gcnt, cb = carry
      nct = ct + gsum[g]
      below = (nct < kt).astype(jnp.int32)
      return nct, gcnt + below, jnp.where(below == 1, nct, cb)

    z = jnp.int32(0)
    _, gstar, cbg = pl.loop(0, ngrp, init_carry=(z, z, z))(gbody)

    cum = plsc.cumsum(totbuf[pl.ds(gstar * L, L)]) + cbg
    m = cum < kt
    b = gstar * L + jnp.sum(m.astype(jnp.int32))
    cb = jnp.maximum(cbg, jnp.max(jnp.where(m, cum, 0)))
    return b, kt - cb

  def process_row(row):
    shrl = lax.shift_right_logical

    zero_hist(L * (NB1 + 1))
    lane_base1 = lane * (NB1 + 1)

    def l1_chunk(buf, carry):
      for sub in range(CL):
        @plsc.parallel_loop(0, NVL, unroll=8)
        def _(i):
          uk = _ukey(buf[sub, pl.ds(i * L, L)])
          plsc.addupdate_scatter(
              hist, [lane_base1 + shrl(uk, B1S)], ones)
      return carry

    stream_pass(row, l1_chunk)
    b1, k1 = scan_level(NB1, jnp.int32(KTOP))

    zero_hist(L * (NB2 + 1))
    lane_base2 = lane * (NB2 + 1)

    def l2_chunk(buf, cntv):
      for sub in range(CL):
        def vbody(i, cntv):
          uk = _ukey(buf[sub, pl.ds(i * L, L)])
          m2 = shrl(uk, B1S) == b1
          plsc.addupdate_scatter(
              hist,
              [lane_base2 + (shrl(uk, B2S) & jnp.int32(NB2 - 1))],
              ones, mask=m2)
          off = jnp.minimum(cntv[0], jnp.int32(CAND - L))
          plsc.store_compressed(cand.at[pl.ds(off, L)], uk, mask=m2)
          return cntv + plsc.all_reduce_population_count(m2)

        cntv = plsc.parallel_loop(0, NVL, unroll=8, carry=cntv)(vbody)
      return cntv

    cntv = stream_pass(row, l2_chunk, carry_init=jnp.zeros((L,), jnp.int32))
    ncand = cntv[0]
    b2, k2 = scan_level(NB2, k1)

    p2 = (b1 << (B1S - B2S)) | b2
    zero_hist(L * (NB3 + 1))
    lane_base3 = lane * (NB3 + 1)

    ncl = jnp.minimum(ncand, jnp.int32(CAND))
    @pl.loop(0, lax.shift_right_logical(ncl + (L - 1), 4))
    def _(i):
      uk = cand[pl.ds(i * L, L)]
      m3 = ((i * L + lane) < ncl) & (shrl(uk, B2S) == p2)
      plsc.addupdate_scatter(
          hist, [lane_base3 + (uk & jnp.int32(NB3 - 1))], ones, mask=m3)

    b3, _ = scan_level(NB3, k2)

    t = (b1 << B1S) | (b2 << B2S) | b3
    mono = ~t
    sgn = shrl(mono, 31)
    ut = jnp.where(sgn == 1, mono ^ jnp.int32(-(2 ** 31)), ~mono)
    thrv = lax.bitcast_convert_type(
        jnp.broadcast_to(ut, (L,)), jnp.float32)

    def mask_chunk(buf, carry):
      for sub in range(CL):
        @plsc.parallel_loop(0, NVL, unroll=8)
        def _(i):
          v = buf[sub, pl.ds(i * L, L)]
          buf[sub, pl.ds(i * L, L)] = jnp.where(
              v >= thrv, v, jnp.float32(0.0))
      return carry

    stream_pass(row, mask_chunk, writeback=True)

  wid = lax.axis_index("c") * NS + lax.axis_index("s")
  for r in range(ROWS_PER_W):
    process_row(wid * ROWS_PER_W + r)


def kernel(features):
  return _topk_mask(features)


# speedup vs baseline: 62.1101x; 1.1000x over previous
import functools

import jax
import jax.numpy as jnp
from jax import lax
from jax.experimental import pallas as pl
from jax.experimental.pallas import tpu as pltpu
from jax.experimental.pallas import tpu_sc as plsc

KTOP = 1024
NC, NS, L = 2, 16, 16
NW = NC * NS
NROW, NL, ND = 64, 32, 4096
ROWS_PER_W = NROW // NW
CL = 8
NCHUNK = NL // CL
NVL = ND // L

B1S, B2S = 21, 10
NB1, NB2, NB3 = 2048, 2048, 1024
CAND = 8192

_mesh = plsc.VectorSubcoreMesh(
    core_axis_name="c", subcore_axis_name="s", num_cores=NC, num_subcores=NS
)


def _ukey(v):
  u = lax.bitcast_convert_type(v, jnp.int32)
  return jnp.where(u < 0, u, u ^ jnp.int32(0x7FFFFFFF))


@functools.partial(
    pl.kernel,
    out_type=jax.ShapeDtypeStruct((NROW, NL, ND), jnp.float32),
    mesh=_mesh,
    scratch_types=[
        [pltpu.VMEM((CL, ND), jnp.float32) for _ in range(2)],
        pltpu.VMEM((NB1,), jnp.int32),
        pltpu.VMEM((NB1,), jnp.int32),
        pltpu.VMEM((CAND,), jnp.int32),
        pltpu.SMEM((256,), jnp.int32),
        [pltpu.SemaphoreType.DMA for _ in range(2)],
        [pltpu.SemaphoreType.DMA for _ in range(2)],
    ],
    compiler_params=pltpu.CompilerParams(needs_layout_passes=False),
)
def _topk_mask(x_hbm, out_hbm, bufs, hist, totbuf, cand, gsum, sin, sout):
  lane = lax.iota(jnp.int32, 16)
  ones = jnp.ones((L,), jnp.int32)

  def in_copy(row, c, k):
    return pltpu.make_async_copy(
        x_hbm.at[row, pl.ds(c * CL, CL)], bufs[k], sin[k])

  def out_copy(row, c, k):
    return pltpu.make_async_copy(
        bufs[k], out_hbm.at[row, pl.ds(c * CL, CL)], sout[k])

  def stream_pass(row, compute_chunk, writeback=False, carry_init=None):
    in_copy(row, 0, 0).start()

    def jbody(j, carry):
      for k in range(2):
        c = 2 * j + k
        other = 1 - k
        if writeback:
          @pl.when(c > 0)
          def _():
            out_copy(row, c - 1, other).wait()

        @pl.when(c + 1 < NCHUNK)
        def _():
          in_copy(row, c + 1, other).start()

        in_copy(row, c, k).wait()
        carry = compute_chunk(bufs[k], carry)
        if writeback:
          out_copy(row, c, k).start()
      return carry

    if carry_init is None:
      pl.loop(0, NCHUNK // 2)(lambda j: jbody(j, None) and None)
      out = None
    else:
      out = pl.loop(0, NCHUNK // 2, init_carry=carry_init)(jbody)

    if writeback:
      out_copy(row, NCHUNK - 1, 1).wait()
    return out

  def zero_hist(nwords):
    zv = jnp.zeros((L,), jnp.int32)

    @plsc.parallel_loop(0, nwords // L, unroll=8)
    def _(i):
      hist[pl.ds(i * L, L)] = zv

  def scan_level(nbins, kt):
    ngrp = nbins // L

    @plsc.parallel_loop(0, ngrp, unroll=2)
    def _(g):
      tot = hist[pl.ds(g * L, L)]
      totbuf[pl.ds(g * L, L)] = tot
      gsum[g] = jnp.sum(tot)

    def gbody(g, carry):
      ct, gcnt, cb = carry
      nct = ct + gsum[g]
      below = (nct < kt).astype(jnp.int32)
      return nct, gcnt + below, jnp.where(below == 1, nct, cb)

    z = jnp.int32(0)
    _, gstar, cbg = pl.loop(0, ngrp, init_carry=(z, z, z))(gbody)

    cum = plsc.cumsum(totbuf[pl.ds(gstar * L, L)]) + cbg
    m = cum < kt
    b = gstar * L + jnp.sum(m.astype(jnp.int32))
    cb = jnp.maximum(cbg, jnp.max(jnp.where(m, cum, 0)))
    return b, kt - cb

  def process_row(row):
    shrl = lax.shift_right_logical

    zero_hist(NB1)

    def l1_chunk(buf, carry):
      for sub in range(CL):
        @plsc.parallel_loop(0, NVL, unroll=8)
        def _(i):
          uk = _ukey(buf[sub, pl.ds(i * L, L)])
          plsc.addupdate_scatter(hist, [shrl(uk, B1S)], ones)
      return carry

    stream_pass(row, l1_chunk)
    b1, k1 = scan_level(NB1, jnp.int32(KTOP))

    zero_hist(NB2)

    def l2_chunk(buf, cntv):
      for sub in range(CL):
        def vbody(i, cntv):
          uk = _ukey(buf[sub, pl.ds(i * L, L)])
          m2 = shrl(uk, B1S) == b1
          plsc.addupdate_scatter(
              hist, [shrl(uk, B2S) & jnp.int32(NB2 - 1)], ones, mask=m2)
          off = jnp.minimum(cntv[0], jnp.int32(CAND - L))
          plsc.store_compressed(cand.at[pl.ds(off, L)], uk, mask=m2)
          return cntv + plsc.all_reduce_population_count(m2)

        cntv = plsc.parallel_loop(0, NVL, unroll=8, carry=cntv)(vbody)
      return cntv

    cntv = stream_pass(row, l2_chunk, carry_init=jnp.zeros((L,), jnp.int32))
    ncand = cntv[0]
    b2, k2 = scan_level(NB2, k1)

    p2 = (b1 << (B1S - B2S)) | b2
    zero_hist(NB3)

    ncl = jnp.minimum(ncand, jnp.int32(CAND))
    @pl.loop(0, lax.shift_right_logical(ncl + (L - 1), 4))
    def _(i):
      uk = cand[pl.ds(i * L, L)]
      m3 = ((i * L + lane) < ncl) & (shrl(uk, B2S) == p2)
      plsc.addupdate_scatter(
          hist, [uk & jnp.int32(NB3 - 1)], ones, mask=m3)

    b3, _ = scan_level(NB3, k2)

    t = (b1 << B1S) | (b2 << B2S) | b3
    mono = ~t
    sgn = shrl(mono, 31)
    ut = jnp.where(sgn == 1, mono ^ jnp.int32(-(2 ** 31)), ~mono)
    thrv = lax.bitcast_convert_type(
        jnp.broadcast_to(ut, (L,)), jnp.float32)

    def mask_chunk(buf, carry):
      for sub in range(CL):
        @plsc.parallel_loop(0, NVL, unroll=8)
        def _(i):
          v = buf[sub, pl.ds(i * L, L)]
          buf[sub, pl.ds(i * L, L)] = jnp.where(
              v >= thrv, v, jnp.float32(0.0))
      return carry

    stream_pass(row, mask_chunk, writeback=True)

  wid = lax.axis_index("c") * NS + lax.axis_index("s")
  for r in range(ROWS_PER_W):
    process_row(wid * ROWS_PER_W + r)


def kernel(features):
  return _topk_mask(features)


# speedup vs baseline: 66.4986x; 1.0707x over previous
import functools

import jax
import jax.numpy as jnp
from jax import lax
from jax.experimental import pallas as pl
from jax.experimental.pallas import tpu as pltpu
from jax.experimental.pallas import tpu_sc as plsc

KTOP = 1024
NC, NS, L = 2, 16, 16
NW = NC * NS
NROW, NL, ND = 64, 32, 4096
ROWS_PER_W = NROW // NW
CL = 8
NCHUNK = NL // CL
NVL = ND // L

B1S, B2S = 21, 10
NB1, NB2, NB3 = 2048, 2048, 1024
CAND = 8192

_mesh = plsc.VectorSubcoreMesh(
    core_axis_name="c", subcore_axis_name="s", num_cores=NC, num_subcores=NS
)


def _ukey(v):
  u = lax.bitcast_convert_type(v, jnp.int32)
  return jnp.where(u < 0, u, u ^ jnp.int32(0x7FFFFFFF))


@functools.partial(
    pl.kernel,
    out_type=jax.ShapeDtypeStruct((NROW, NL, ND), jnp.float32),
    mesh=_mesh,
    scratch_types=[
        [pltpu.VMEM((CL, ND), jnp.float32) for _ in range(2)],
        pltpu.VMEM((NB1,), jnp.int32),
        pltpu.VMEM((NB1,), jnp.int32),
        pltpu.VMEM((CAND,), jnp.int32),
        pltpu.SMEM((256,), jnp.int32),
        [pltpu.SemaphoreType.DMA for _ in range(2)],
        [pltpu.SemaphoreType.DMA for _ in range(2)],
    ],
    compiler_params=pltpu.CompilerParams(needs_layout_passes=False),
)
def _topk_mask(x_hbm, out_hbm, bufs, hist, totbuf, cand, gsum, sin, sout):
  lane = lax.iota(jnp.int32, 16)
  ones = jnp.ones((L,), jnp.int32)

  def in_copy(row, c, k):
    return pltpu.make_async_copy(
        x_hbm.at[row, pl.ds(c * CL, CL)], bufs[k], sin[k])

  def out_copy(row, c, k):
    return pltpu.make_async_copy(
        bufs[k], out_hbm.at[row, pl.ds(c * CL, CL)], sout[k])

  def stream_pass(row, compute_chunk, writeback=False, carry_init=None,
                  primed=False):
    if not primed:
      in_copy(row, 0, 0).start()

    def jbody(j, carry):
      for k in range(2):
        c = 2 * j + k
        other = 1 - k
        if writeback:
          @pl.when(c > 0)
          def _():
            out_copy(row, c - 1, other).wait()

        @pl.when(c + 1 < NCHUNK)
        def _():
          in_copy(row, c + 1, other).start()

        in_copy(row, c, k).wait()
        carry = compute_chunk(bufs[k], carry)
        if writeback:
          out_copy(row, c, k).start()
      return carry

    if carry_init is None:
      pl.loop(0, NCHUNK // 2)(lambda j: jbody(j, None) and None)
      out = None
    else:
      out = pl.loop(0, NCHUNK // 2, init_carry=carry_init)(jbody)

    if writeback:
      out_copy(row, NCHUNK - 1, 1).wait()
    return out

  def zero_hist(nwords):
    zv = jnp.zeros((L,), jnp.int32)

    @plsc.parallel_loop(0, nwords // L, unroll=8)
    def _(i):
      hist[pl.ds(i * L, L)] = zv

  def scan_level(nbins, kt):
    ngrp = nbins // L

    @plsc.parallel_loop(0, ngrp, unroll=2)
    def _(g):
      tot = hist[pl.ds(g * L, L)]
      totbuf[pl.ds(g * L, L)] = tot
      gsum[g] = jnp.sum(tot)

    def gbody(g, carry):
      ct, gcnt, cb = carry
      nct = ct + gsum[g]
      below = (nct < kt).astype(jnp.int32)
      return nct, gcnt + below, jnp.where(below == 1, nct, cb)

    z = jnp.int32(0)
    _, gstar, cbg = pl.loop(0, ngrp, init_carry=(z, z, z))(gbody)

    cum = plsc.cumsum(totbuf[pl.ds(gstar * L, L)]) + cbg
    m = cum < kt
    b = gstar * L + jnp.sum(m.astype(jnp.int32))
    cb = jnp.maximum(cbg, jnp.max(jnp.where(m, cum, 0)))
    return b, kt - cb

  def process_row(row):
    shrl = lax.shift_right_logical

    zero_hist(NB1)

    def l1_chunk(buf, carry):
      for sub in range(CL):
        @plsc.parallel_loop(0, NVL, unroll=16)
        def _(i):
          uk = _ukey(buf[sub, pl.ds(i * L, L)])
          plsc.addupdate_scatter(hist, [shrl(uk, B1S)], ones)
      return carry

    stream_pass(row, l1_chunk)
    in_copy(row, 0, 0).start()
    b1, k1 = scan_level(NB1, jnp.int32(KTOP))

    zero_hist(NB2)

    def l2_chunk(buf, cntv):
      for sub in range(CL):
        def vbody(i, cntv):
          uk = _ukey(buf[sub, pl.ds(i * L, L)])
          m2 = shrl(uk, B1S) == b1
          plsc.addupdate_scatter(
              hist, [shrl(uk, B2S) & jnp.int32(NB2 - 1)], ones, mask=m2)
          off = jnp.minimum(cntv[0], jnp.int32(CAND - L))
          plsc.store_compressed(cand.at[pl.ds(off, L)], uk, mask=m2)
          return cntv + plsc.all_reduce_population_count(m2)

        cntv = plsc.parallel_loop(0, NVL, unroll=8, carry=cntv)(vbody)
      return cntv

    cntv = stream_pass(row, l2_chunk, carry_init=jnp.zeros((L,), jnp.int32),
                       primed=True)
    in_copy(row, 0, 0).start()
    ncand = cntv[0]
    b2, k2 = scan_level(NB2, k1)

    p2 = (b1 << (B1S - B2S)) | b2
    zero_hist(NB3)

    ncl = jnp.minimum(ncand, jnp.int32(CAND))
    @pl.loop(0, lax.shift_right_logical(ncl + (L - 1), 4))
    def _(i):
      uk = cand[pl.ds(i * L, L)]
      m3 = ((i * L + lane) < ncl) & (shrl(uk, B2S) == p2)
      plsc.addupdate_scatter(
          hist, [uk & jnp.int32(NB3 - 1)], ones, mask=m3)

    b3, _ = scan_level(NB3, k2)

    t = (b1 << B1S) | (b2 << B2S) | b3
    mono = ~t
    sgn = shrl(mono, 31)
    ut = jnp.where(sgn == 1, mono ^ jnp.int32(-(2 ** 31)), ~mono)
    thrv = lax.bitcast_convert_type(
        jnp.broadcast_to(ut, (L,)), jnp.float32)

    def mask_chunk(buf, carry):
      for sub in range(CL):
        @plsc.parallel_loop(0, NVL, unroll=16)
        def _(i):
          v = buf[sub, pl.ds(i * L, L)]
          buf[sub, pl.ds(i * L, L)] = jnp.where(
              v >= thrv, v, jnp.float32(0.0))
      return carry

    stream_pass(row, mask_chunk, writeback=True, primed=True)

  wid = lax.axis_index("c") * NS + lax.axis_index("s")
  for r in range(ROWS_PER_W):
    process_row(wid * ROWS_PER_W + r)


def kernel(features):
  return _topk_mask(features)


# speedup vs baseline: 66.5059x; 1.0001x over previous
import functools

import jax
import jax.numpy as jnp
from jax import lax
from jax.experimental import pallas as pl
from jax.experimental.pallas import tpu as pltpu
from jax.experimental.pallas import tpu_sc as plsc

KTOP = 1024
NC, NS, L = 2, 16, 16
NW = NC * NS
NROW, NL, ND = 64, 32, 4096
ROWS_PER_W = NROW // NW
CL = 8
NCHUNK = NL // CL
NVL = ND // L

B1S, B2S = 21, 10
NB1, NB2, NB3 = 2048, 2048, 1024
CAND = 8192

_mesh = plsc.VectorSubcoreMesh(
    core_axis_name="c", subcore_axis_name="s", num_cores=NC, num_subcores=NS
)


def _ukey(v):
  u = lax.bitcast_convert_type(v, jnp.int32)
  return jnp.where(u < 0, u, u ^ jnp.int32(0x7FFFFFFF))


@functools.partial(
    pl.kernel,
    out_type=jax.ShapeDtypeStruct((NROW, NL, ND), jnp.float32),
    mesh=_mesh,
    scratch_types=[
        [pltpu.VMEM((CL, ND), jnp.float32) for _ in range(2)],
        pltpu.VMEM((NB1,), jnp.int32),
        pltpu.VMEM((NB1,), jnp.int32),
        pltpu.VMEM((CAND,), jnp.int32),
        pltpu.SMEM((256,), jnp.int32),
        [pltpu.SemaphoreType.DMA for _ in range(2)],
        [pltpu.SemaphoreType.DMA for _ in range(2)],
    ],
    compiler_params=pltpu.CompilerParams(needs_layout_passes=False),
)
def _topk_mask(x_hbm, out_hbm, bufs, hist, totbuf, cand, gsum, sin, sout):
  lane = lax.iota(jnp.int32, 16)
  ones = jnp.ones((L,), jnp.int32)

  def in_copy(row, c, k):
    return pltpu.make_async_copy(
        x_hbm.at[row, pl.ds(c * CL, CL)], bufs[k], sin[k])

  def out_copy(row, c, k):
    return pltpu.make_async_copy(
        bufs[k], out_hbm.at[row, pl.ds(c * CL, CL)], sout[k])

  def stream_pass(row, compute_chunk, writeback=False, carry_init=None,
                  primed=False):
    if not primed:
      in_copy(row, 0, 0).start()

    def jbody(j, carry):
      for k in range(2):
        c = 2 * j + k
        other = 1 - k
        if writeback:
          @pl.when(c > 0)
          def _():
            out_copy(row, c - 1, other).wait()

        @pl.when(c + 1 < NCHUNK)
        def _():
          in_copy(row, c + 1, other).start()

        in_copy(row, c, k).wait()
        carry = compute_chunk(bufs[k], carry)
        if writeback:
          out_copy(row, c, k).start()
      return carry

    if carry_init is None:
      pl.loop(0, NCHUNK // 2)(lambda j: jbody(j, None) and None)
      out = None
    else:
      out = pl.loop(0, NCHUNK // 2, init_carry=carry_init)(jbody)

    if writeback:
      out_copy(row, NCHUNK - 1, 1).wait()
    return out

  def zero_hist(nwords):
    zv = jnp.zeros((L,), jnp.int32)

    @plsc.parallel_loop(0, nwords // L, unroll=8)
    def _(i):
      hist[pl.ds(i * L, L)] = zv

  def scan_level(nbins, kt):
    ngrp = nbins // L

    @plsc.parallel_loop(0, ngrp, unroll=2)
    def _(g):
      tot = hist[pl.ds(g * L, L)]
      totbuf[pl.ds(g * L, L)] = tot
      gsum[g] = jnp.sum(tot)

    def gbody(g, carry):
      ct, gcnt, cb = carry
      nct = ct + gsum[g]
      below = (nct < kt).astype(jnp.int32)
      return nct, gcnt + below, jnp.where(below == 1, nct, cb)

    z = jnp.int32(0)
    _, gstar, cbg = pl.loop(0, ngrp, init_carry=(z, z, z))(gbody)

    cum = plsc.cumsum(totbuf[pl.ds(gstar * L, L)]) + cbg
    m = cum < kt
    b = gstar * L + jnp.sum(m.astype(jnp.int32))
    cb = jnp.maximum(cbg, jnp.max(jnp.where(m, cum, 0)))
    return b, kt - cb

  def process_row(row, primed_l1=False):
    shrl = lax.shift_right_logical

    zero_hist(NB1)

    def l1_chunk(buf, carry):
      for sub in range(CL):
        @plsc.parallel_loop(0, NVL, unroll=16)
        def _(i):
          uk = _ukey(buf[sub, pl.ds(i * L, L)])
          plsc.addupdate_scatter(hist, [shrl(uk, B1S)], ones)
      return carry

    stream_pass(row, l1_chunk, primed=primed_l1)
    in_copy(row, 0, 0).start()
    b1, k1 = scan_level(NB1, jnp.int32(KTOP))

    zero_hist(NB2)

    def l2_chunk(buf, cntv):
      for sub in range(CL):
        def vbody(i, cntv):
          uk = _ukey(buf[sub, pl.ds(i * L, L)])
          m2 = shrl(uk, B1S) == b1
          plsc.addupdate_scatter(
              hist, [shrl(uk, B2S) & jnp.int32(NB2 - 1)], ones, mask=m2)
          off = jnp.minimum(cntv[0], jnp.int32(CAND - L))
          plsc.store_compressed(cand.at[pl.ds(off, L)], uk, mask=m2)
          return cntv + plsc.all_reduce_population_count(m2)

        cntv = plsc.parallel_loop(0, NVL, unroll=8, carry=cntv)(vbody)
      return cntv

    cntv = stream_pass(row, l2_chunk, carry_init=jnp.zeros((L,), jnp.int32),
                       primed=True)
    in_copy(row, 0, 0).start()
    ncand = cntv[0]
    b2, k2 = scan_level(NB2, k1)

    p2 = (b1 << (B1S - B2S)) | b2
    zero_hist(NB3)

    ncl = jnp.minimum(ncand, jnp.int32(CAND))
    @pl.loop(0, lax.shift_right_logical(ncl + (L - 1), 4))
    def _(i):
      uk = cand[pl.ds(i * L, L)]
      m3 = ((i * L + lane) < ncl) & (shrl(uk, B2S) == p2)
      plsc.addupdate_scatter(
          hist, [uk & jnp.int32(NB3 - 1)], ones, mask=m3)

    b3, _ = scan_level(NB3, k2)

    t = (b1 << B1S) | (b2 << B2S) | b3
    mono = ~t
    sgn = shrl(mono, 31)
    ut = jnp.where(sgn == 1, mono ^ jnp.int32(-(2 ** 31)), ~mono)
    thrv = lax.bitcast_convert_type(
        jnp.broadcast_to(ut, (L,)), jnp.float32)

    def mask_chunk(buf, carry):
      for sub in range(CL):
        @plsc.parallel_loop(0, NVL, unroll=16)
        def _(i):
          v = buf[sub, pl.ds(i * L, L)]
          buf[sub, pl.ds(i * L, L)] = jnp.where(
              v >= thrv, v, jnp.float32(0.0))
      return carry

    stream_pass(row, mask_chunk, writeback=True, primed=True)

  wid = lax.axis_index("c") * NS + lax.axis_index("s")
  for r in range(ROWS_PER_W):
    row = wid * ROWS_PER_W + r
    if r + 1 < ROWS_PER_W:
      process_row(row, primed_l1=(r > 0))
      in_copy(row + 1, 0, 0).start()
    else:
      process_row(row, primed_l1=True)


def kernel(features):
  return _topk_mask(features)
